# baseline (speedup 1.0000x reference)
import functools

import jax
import jax.numpy as jnp
from jax import lax
from jax.experimental import pallas as pl
from jax.experimental.pallas import tpu as pltpu
from jax.experimental.pallas import tpu_sc as plsc

D_E = 128
NUM_WORKERS = 32


def _sc_gather(idx3, table, per_w, seq):
    n_batch = NUM_WORKERS * per_w
    mesh = plsc.VectorSubcoreMesh(core_axis_name="c", subcore_axis_name="s")

    @functools.partial(
        pl.kernel,
        out_type=jax.ShapeDtypeStruct((n_batch, seq, D_E), jnp.float32),
        mesh=mesh,
        scratch_types=[
            pltpu.VMEM((per_w, seq), jnp.int32),
            pltpu.VMEM((2, seq, D_E), jnp.float32),
            pltpu.SemaphoreType.DMA,
            pltpu.SemaphoreType.DMA,
        ],
    )
    def k(idx_hbm, table_hbm, out_hbm, idx_v, rows_v, g0, g1):
        assert per_w % 2 == 0
        wid = lax.axis_index("s") * 2 + lax.axis_index("c")
        base = wid * per_w
        pltpu.sync_copy(idx_hbm.at[wid], idx_v)

        pltpu.async_copy(table_hbm.at[idx_v.at[0]], rows_v.at[0], g0)

        def body(i, _):
            b = i * 2
            pltpu.async_copy(table_hbm.at[idx_v.at[b + 1]], rows_v.at[1], g1)
            pltpu.make_async_copy(
                table_hbm.at[idx_v.at[b]], rows_v.at[0], g0
            ).wait()
            pltpu.sync_copy(rows_v.at[0], out_hbm.at[base + b])

            @pl.when(b + 2 < per_w)
            def _():
                pltpu.async_copy(
                    table_hbm.at[idx_v.at[b + 2]], rows_v.at[0], g0
                )

            pltpu.make_async_copy(
                table_hbm.at[idx_v.at[b + 1]], rows_v.at[1], g1
            ).wait()
            pltpu.sync_copy(rows_v.at[1], out_hbm.at[base + b + 1])
            return 0

        lax.fori_loop(0, per_w // 2, body, 0, unroll=False)

    return k(idx3, table)


N_PARTS = 4


def kernel(input_ids, table):
    b, s = input_ids.shape
    bp = b // N_PARTS
    per_w = bp // NUM_WORKERS
    assert per_w * NUM_WORKERS * N_PARTS == b
    parts = []
    for p in range(N_PARTS):
        ids_p = lax.slice_in_dim(input_ids, p * bp, (p + 1) * bp, axis=0)
        idx3 = ids_p.reshape(NUM_WORKERS, per_w, s).astype(jnp.int32)
        parts.append(_sc_gather(idx3, table, per_w, s))
    return jnp.concatenate(parts, axis=0)


# speedup vs baseline: 1.6921x; 1.6921x over previous
import functools

import jax
import jax.numpy as jnp
from jax import lax
from jax.experimental import pallas as pl
from jax.experimental.pallas import tpu as pltpu
from jax.experimental.pallas import tpu_sc as plsc

D_E = 128
NUM_WORKERS = 32


def _sc_gather(idx3, table, per_w, seq):
    n_batch = NUM_WORKERS * per_w
    mesh = plsc.VectorSubcoreMesh(core_axis_name="c", subcore_axis_name="s")

    @functools.partial(
        pl.kernel,
        out_type=jax.ShapeDtypeStruct((n_batch, seq, D_E), jnp.float32),
        mesh=mesh,
        compiler_params=pltpu.CompilerParams(use_tc_tiling_on_sc=True),
        scratch_types=[
            pltpu.VMEM((per_w, 128), jnp.int32),
            pltpu.VMEM((2, seq, D_E), jnp.float32),
            pltpu.SemaphoreType.DMA,
            pltpu.SemaphoreType.DMA,
        ],
    )
    def k(idx_hbm, table_hbm, out_hbm, idx_v, rows_v, g0, g1):
        assert per_w % 2 == 0
        wid = lax.axis_index("s") * 2 + lax.axis_index("c")
        base = wid * per_w
        pltpu.sync_copy(idx_hbm.at[wid], idx_v)

        def gidx(b):
            return idx_v.at[b].at[pl.ds(0, seq)]

        pltpu.async_copy(table_hbm.at[gidx(0)], rows_v.at[0], g0)

        def body(i, _):
            b = i * 2
            pltpu.async_copy(table_hbm.at[gidx(b + 1)], rows_v.at[1], g1)
            pltpu.make_async_copy(
                table_hbm.at[gidx(b)], rows_v.at[0], g0
            ).wait()
            pltpu.sync_copy(rows_v.at[0], out_hbm.at[base + b])

            @pl.when(b + 2 < per_w)
            def _():
                pltpu.async_copy(table_hbm.at[gidx(b + 2)], rows_v.at[0], g0)

            pltpu.make_async_copy(
                table_hbm.at[gidx(b + 1)], rows_v.at[1], g1
            ).wait()
            pltpu.sync_copy(rows_v.at[1], out_hbm.at[base + b + 1])
            return 0

        lax.fori_loop(0, per_w // 2, body, 0, unroll=False)

    return k(idx3, table)


def kernel(input_ids, table):
    b, s = input_ids.shape
    per_w = b // NUM_WORKERS
    assert per_w * NUM_WORKERS == b
    ids = input_ids.astype(jnp.int32)
    ids = jnp.pad(ids, ((0, 0), (0, 128 - s)))
    idx3 = ids.reshape(NUM_WORKERS, per_w, 128)
    return _sc_gather(idx3, table, per_w, s)


# speedup vs baseline: 1.9784x; 1.1692x over previous
import functools

import jax
import jax.numpy as jnp
from jax import lax
from jax.experimental import pallas as pl
from jax.experimental.pallas import tpu as pltpu
from jax.experimental.pallas import tpu_sc as plsc

D_E = 128
NUM_WORKERS = 32
NBUF = 4


def _sc_gather(idx2d, table, per_w, seq):
    n_batch = NUM_WORKERS * per_w
    n_chunks = per_w // 2
    chunk_idx = 2 * seq
    mesh = plsc.VectorSubcoreMesh(core_axis_name="c", subcore_axis_name="s")

    @functools.partial(
        pl.kernel,
        out_type=jax.ShapeDtypeStruct((n_batch, seq, D_E), jnp.float32),
        mesh=mesh,
        scratch_types=[
            pltpu.VMEM((n_chunks, 128), jnp.int32),
            pltpu.VMEM((NBUF, chunk_idx, D_E), jnp.float32),
        ]
        + [pltpu.SemaphoreType.DMA] * (2 * NBUF),
    )
    def k(idx_hbm, table_hbm, out_hbm, idx_v, rows_v, *sems):
        gs = sems[:NBUF]
        ws = sems[NBUF:]
        wid = lax.axis_index("s") * 2 + lax.axis_index("c")
        base_b = wid * per_w
        base_c = wid * n_chunks
        pltpu.sync_copy(idx_hbm.at[pl.ds(base_c, n_chunks)], idx_v)

        def gather(c, r):
            return pltpu.async_copy(
                table_hbm.at[idx_v.at[c].at[pl.ds(0, chunk_idx)]],
                rows_v.at[r],
                gs[r],
            )

        def wait_gather(c, r):
            pltpu.make_async_copy(
                table_hbm.at[idx_v.at[c].at[pl.ds(0, chunk_idx)]],
                rows_v.at[r],
                gs[r],
            ).wait()

        def write(c, r):
            for h in range(2):
                pltpu.async_copy(
                    rows_v.at[r].at[pl.ds(h * seq, seq)],
                    out_hbm.at[base_b + 2 * c + h],
                    ws[r],
                )

        def wait_write(c, r):
            for h in range(2):
                pltpu.make_async_copy(
                    rows_v.at[r].at[pl.ds(h * seq, seq)],
                    out_hbm.at[base_b + 2 * c + h],
                    ws[r],
                ).wait()

        for r in range(NBUF):
            gather(r, r)

        def body(i, _):
            for r in range(NBUF):
                c = i * NBUF + r
                wait_gather(c, r)
                write(c, r)

                @pl.when(c + NBUF < n_chunks)
                def _():
                    wait_write(c, r)
                    gather(c + NBUF, r)

            return 0

        lax.fori_loop(0, n_chunks // NBUF, body, 0, unroll=False)

        for r in range(NBUF):
            wait_write(n_chunks - NBUF + r, r)

    return k(idx2d, table)


def kernel(input_ids, table):
    b, s = input_ids.shape
    per_w = b // NUM_WORKERS
    assert per_w * NUM_WORKERS == b and per_w % (2 * NBUF) == 0
    ids = input_ids.astype(jnp.int32).reshape(b // 2, 2 * s)
    ids = jnp.pad(ids, ((0, 0), (0, 128 - 2 * s)))
    return _sc_gather(ids, table, per_w, s)


# speedup vs baseline: 1.9985x; 1.0101x over previous
import functools

import jax
import jax.numpy as jnp
from jax import lax
from jax.experimental import pallas as pl
from jax.experimental.pallas import tpu as pltpu
from jax.experimental.pallas import tpu_sc as plsc

D_E = 128
NUM_WORKERS = 32
NBUF = 8


def _sc_gather(idx2d, table, per_w, seq):
    n_batch = NUM_WORKERS * per_w
    n_chunks = per_w // 2
    chunk_idx = 2 * seq
    mesh = plsc.VectorSubcoreMesh(core_axis_name="c", subcore_axis_name="s")

    @functools.partial(
        pl.kernel,
        out_type=jax.ShapeDtypeStruct((n_batch, seq, D_E), jnp.float32),
        mesh=mesh,
        scratch_types=[
            pltpu.VMEM((n_chunks, 128), jnp.int32),
            pltpu.VMEM((NBUF, chunk_idx, D_E), jnp.float32),
        ]
        + [pltpu.SemaphoreType.DMA] * (2 * NBUF),
    )
    def k(idx_hbm, table_hbm, out_hbm, idx_v, rows_v, *sems):
        gs = sems[:NBUF]
        ws = sems[NBUF:]
        wid = lax.axis_index("s") * 2 + lax.axis_index("c")
        base_b = wid * per_w
        base_c = wid * n_chunks
        pltpu.sync_copy(idx_hbm.at[pl.ds(base_c, n_chunks)], idx_v)

        def gather(c, r):
            return pltpu.async_copy(
                table_hbm.at[idx_v.at[c].at[pl.ds(0, chunk_idx)]],
                rows_v.at[r],
                gs[r],
            )

        def wait_gather(c, r):
            pltpu.make_async_copy(
                table_hbm.at[idx_v.at[c].at[pl.ds(0, chunk_idx)]],
                rows_v.at[r],
                gs[r],
            ).wait()

        def write(c, r):
            for h in range(2):
                pltpu.async_copy(
                    rows_v.at[r].at[pl.ds(h * seq, seq)],
                    out_hbm.at[base_b + 2 * c + h],
                    ws[r],
                )

        def wait_write(c, r):
            for h in range(2):
                pltpu.make_async_copy(
                    rows_v.at[r].at[pl.ds(h * seq, seq)],
                    out_hbm.at[base_b + 2 * c + h],
                    ws[r],
                ).wait()

        for r in range(NBUF):
            gather(r, r)

        def body(i, _):
            for r in range(NBUF):
                c = i * NBUF + r
                wait_gather(c, r)
                write(c, r)

                @pl.when(c + NBUF < n_chunks)
                def _():
                    wait_write(c, r)
                    gather(c + NBUF, r)

            return 0

        lax.fori_loop(0, n_chunks // NBUF, body, 0, unroll=False)

        for r in range(NBUF):
            wait_write(n_chunks - NBUF + r, r)

    return k(idx2d, table)


def kernel(input_ids, table):
    b, s = input_ids.shape
    per_w = b // NUM_WORKERS
    assert per_w * NUM_WORKERS == b and per_w % (2 * NBUF) == 0
    ids = input_ids.astype(jnp.int32).reshape(b // 2, 2 * s)
    ids = jnp.pad(ids, ((0, 0), (0, 128 - 2 * s)))
    return _sc_gather(ids, table, per_w, s)
